# baseline (speedup 1.0000x reference)
import functools

import jax
import jax.numpy as jnp
from jax import lax
from jax.experimental import pallas as pl
from jax.experimental.pallas import tpu as pltpu
from jax.experimental.pallas import tpu_sc as plsc

NC = 2
NS = 16
NW = NC * NS
L = 16
W = 128

R = 16


def _emb_call(n_rows, d, s_len):
    c_sub = d // W
    rs = R * c_sub
    rows_per_w = n_rows // NW
    n_blocks = rows_per_w // R
    mesh = plsc.VectorSubcoreMesh(core_axis_name="c", subcore_axis_name="s")

    @functools.partial(
        pl.kernel,
        out_type=jax.ShapeDtypeStruct((n_rows * c_sub, W), jnp.float32),
        mesh=mesh,
        scratch_types=[
            pltpu.VMEM((1, R), jnp.int32),
            pltpu.VMEM((1, rs), jnp.int32),
            pltpu.VMEM((rs, W), jnp.float32),
            pltpu.VMEM((c_sub, W), jnp.float32),
        ],
    )
    def run(ids_hbm, word_hbm, pos_hbm, out_hbm, idx_v, idx2_v, buf, w0_v):
        c = lax.axis_index("c")
        s = lax.axis_index("s")
        wid = s * NC + c
        base = wid * rows_per_w
        pbase = lax.rem(base, s_len)

        pltpu.sync_copy(word_hbm.at[pl.ds(0, c_sub), :], w0_v)

        def block(i, _):
            rb = base + i * R
            pb = pbase + i * R
            pltpu.sync_copy(ids_hbm.at[pl.ds(rb, R)], idx_v.at[0])
            iota = lax.iota(jnp.int32, L)
            rows_per_chunk = L // c_sub
            shift = c_sub.bit_length() - 1
            jpat = jnp.bitwise_and(iota, c_sub - 1)
            rpat = lax.shift_right_logical(iota, shift)
            for g2 in range(rs // L):
                r0 = g2 * rows_per_chunk
                v = idx_v[0, pl.ds((r0 // L) * L, L)]
                chunk = jpat
                for t in range(rows_per_chunk):
                    a = v[r0 % L + t]
                    sel = rpat == t
                    chunk = jnp.where(sel, a * c_sub + jpat, chunk)
                idx2_v[0, pl.ds(g2 * L, L)] = chunk
            pltpu.sync_copy(pos_hbm.at[pl.ds(pb * c_sub, rs), :], buf)
            pltpu.sync_copy(word_hbm.at[idx2_v.at[0]], buf, add=True)

            for g in range(R // L):
                v = idx_v[0, pl.ds(g * L, L)]
                for r in range(L):
                    @pl.when(v[r] == 0)
                    def _row(r=r, g=g):
                        row = (g * L + r) * c_sub
                        for j in range(c_sub):
                            def sub_h(h, _, j=j):
                                sl = pl.ds(h * L, L)
                                buf[row + j, sl] = buf[row + j, sl] - w0_v[j, sl]
                                return ()
                            lax.fori_loop(0, W // L, sub_h, (), unroll=True)

            pltpu.sync_copy(buf, out_hbm.at[pl.ds(rb * c_sub, rs), :])
            return ()

        lax.fori_loop(0, n_blocks, block, ())

    return run


def kernel(input_ids, word_table, pos_table):
    b, s_len = input_ids.shape
    v, d = word_table.shape
    ids = input_ids.reshape(-1).astype(jnp.int32)
    word2 = word_table.reshape(v * (d // W), W)
    pos2 = pos_table.reshape(s_len * (d // W), W)
    out = _emb_call(b * s_len, d, s_len)(ids, word2, pos2)
    return out.reshape(b, s_len, d)
